# baseline (speedup 1.0000x reference)
import functools
import jax
import jax.numpy as jnp
from jax import lax
from jax.experimental import pallas as pl
from jax.experimental.pallas import tpu as pltpu, tpu_sc as plsc

N = 10000
E = 160000
D_IN = 128
HID = 64
HEADS = 8
NC_OUT = 4

NP = 10240
EP = 163840
PAD_NODE = N

NUM_TILES = 32
EDGES_PER_TILE = EP // NUM_TILES
CHUNK = 128
NCHUNKS = EDGES_PER_TILE // CHUNK
CHUNK_B = 64
NCHUNKS_B = EDGES_PER_TILE // CHUNK_B
ROWS_PER_TILE = NP // 16

BN = 256
GRID_N = NP // BN


def _tc1_body(xq, xa, wsqa, asqa, wdqa, adqa, wsaq, asaq, wdaq, adaq,
              hs_qa, al_s_qa, al_d_qa, hs_aq, al_s_aq, al_d_aq):
    xqb = xq[...]
    xab = xa[...]
    hq = jnp.dot(xqb, wsqa[...], preferred_element_type=jnp.float32)
    hs_qa[...] = hq.astype(jnp.bfloat16)
    al_s_qa[...] = jnp.dot(hq, asqa[...], preferred_element_type=jnp.float32)
    vd_qa = jnp.dot(wdqa[...], adqa[...], preferred_element_type=jnp.float32)
    al_d_qa[...] = jnp.dot(xab, vd_qa, preferred_element_type=jnp.float32)
    ha = jnp.dot(xab, wsaq[...], preferred_element_type=jnp.float32)
    hs_aq[...] = ha.astype(jnp.bfloat16)
    al_s_aq[...] = jnp.dot(ha, asaq[...], preferred_element_type=jnp.float32)
    vd_aq = jnp.dot(wdaq[...], adaq[...], preferred_element_type=jnp.float32)
    al_d_aq[...] = jnp.dot(xqb, vd_aq, preferred_element_type=jnp.float32)


def _sc_phase_a_body(epa_qa, epa_aq, asq, adq, asa, ada, z16,
                     ee_qa, ee_aq, dp_qa, dp_aq,
                     idx4, aa4, bb4, ee4, den_sh,
                     sa0, sa1, sa2, sa3, sb0, sb1, sb2, sb3,
                     sw0, sw1, sw2, sw3, ss0, ss1, ss2, ss3):
    cid = lax.axis_index("c")
    sid = lax.axis_index("s")
    wid = cid * 16 + sid
    r0 = sid * ROWS_PER_TILE
    sem_a = (sa0, sa1, sa2, sa3)
    sem_b = (sb0, sb1, sb2, sb3)
    sem_w = (sw0, sw1, sw2, sw3)
    sem_s = (ss0, ss1, ss2, ss3)

    for epa, asrc, adst, ee_o, dp_o in (
        (epa_qa, asq, adq, ee_qa, dp_qa),
        (epa_aq, asa, ada, ee_aq, dp_aq),
    ):
        pltpu.sync_copy(z16.at[pl.ds(r0, ROWS_PER_TILE)],
                        den_sh.at[pl.ds(r0, ROWS_PER_TILE)])
        plsc.subcore_barrier()

        def fire(k, b, epa=epa, asrc=asrc, adst=adst):
            g = wid * NCHUNKS + k
            pltpu.sync_copy(epa.at[g], idx4.at[b])
            pltpu.make_async_copy(asrc.at[idx4.at[b].at[0]], aa4.at[b],
                                  sem_a[b]).start()
            pltpu.make_async_copy(adst.at[idx4.at[b].at[1]], bb4.at[b],
                                  sem_b[b]).start()

        def drain(k, b, ee_o=ee_o):
            base = (wid * NCHUNKS + k) * CHUNK
            pltpu.make_async_copy(ee4.at[b], ee_o.at[pl.ds(base, CHUNK)],
                                  sem_w[b]).wait()
            pltpu.make_async_copy(ee4.at[b], den_sh.at[idx4.at[b].at[1]],
                                  sem_s[b]).wait()

        def process(k, b, asrc=asrc, adst=adst, ee_o=ee_o):
            base = (wid * NCHUNKS + k) * CHUNK
            pltpu.make_async_copy(asrc.at[idx4.at[b].at[0]], aa4.at[b],
                                  sem_a[b]).wait()
            pltpu.make_async_copy(adst.at[idx4.at[b].at[1]], bb4.at[b],
                                  sem_b[b]).wait()

            @pl.when(k > 0)
            def _():
                drain(k - 1, (b + 3) % 4)

            @pl.when(k + 3 < NCHUNKS)
            def _():
                fire(k + 3, (b + 3) % 4)

            def row(i, _):
                sv = aa4[b, i, :] + bb4[b, i, :]
                e = jnp.maximum(sv, 0.2 * sv)
                ee4[b, i, :] = jnp.exp(e)
                return 0
            lax.fori_loop(0, CHUNK, row, 0)
            pltpu.make_async_copy(ee4.at[b], ee_o.at[pl.ds(base, CHUNK)],
                                  sem_w[b]).start()
            pltpu.make_async_copy(ee4.at[b], den_sh.at[idx4.at[b].at[1]],
                                  sem_s[b]).start(add=True)

        fire(0, 0)
        fire(1, 1)
        fire(2, 2)

        def quad(kk, _):
            for b in range(4):
                process(4 * kk + b, b)
            return 0
        lax.fori_loop(0, NCHUNKS // 4, quad, 0)
        drain(NCHUNKS - 1, 3)

        plsc.subcore_barrier()
        pltpu.sync_copy(den_sh.at[pl.ds(r0, ROWS_PER_TILE)],
                        dp_o.at[cid, pl.ds(r0, ROWS_PER_TILE)])
        plsc.subcore_barrier()


def _sc_phase_b_body(ep_qa, ep_aq, ee_qa, ee_aq, den_qa, den_aq,
                     hs_qa, hs_aq, z64,
                     op_qa, op_aq,
                     idx4, ee2, den2, wbuf, hs4, msg2, out_sh,
                     se0, se1, sd0, sd1, sh0, sh1, sh2, sh3, ss0, ss1):
    cid = lax.axis_index("c")
    sid = lax.axis_index("s")
    wid = cid * 16 + sid
    r0 = sid * ROWS_PER_TILE
    sem_e = (se0, se1)
    sem_d = (sd0, sd1)
    sem_h = (sh0, sh1, sh2, sh3)
    sem_s = (ss0, ss1)

    for ep, ee, den, hs, op_o in (
        (ep_qa, ee_qa, den_qa, hs_qa, op_qa),
        (ep_aq, ee_aq, den_aq, hs_aq, op_aq),
    ):
        pltpu.sync_copy(z64.at[pl.ds(r0, ROWS_PER_TILE)],
                        out_sh.at[pl.ds(r0, ROWS_PER_TILE)])
        plsc.subcore_barrier()

        def fire_far(k, b, ep=ep, hs=hs):
            g = wid * NCHUNKS_B + k
            pltpu.sync_copy(ep.at[g], idx4.at[b])
            pltpu.make_async_copy(hs.at[idx4.at[b].at[0]], hs4.at[b],
                                  sem_h[b]).start()

        def fire_near(k, b, b2, ee=ee, den=den):
            base = (wid * NCHUNKS_B + k) * CHUNK_B
            pltpu.make_async_copy(den.at[idx4.at[b].at[1]], den2.at[b2],
                                  sem_d[b2]).start()
            pltpu.make_async_copy(ee.at[pl.ds(base, CHUNK_B)], ee2.at[b2],
                                  sem_e[b2]).start()

        def wait_scatter(b, b2):
            pltpu.make_async_copy(msg2.at[b2], out_sh.at[idx4.at[b].at[1]],
                                  sem_s[b2]).wait()

        def process(k, b, b2, ee=ee, den=den, hs=hs):
            pltpu.make_async_copy(den.at[idx4.at[b].at[1]], den2.at[b2],
                                  sem_d[b2]).wait()
            pltpu.make_async_copy(ee.at[pl.ds(0, CHUNK_B)], ee2.at[b2],
                                  sem_e[b2]).wait()

            def wrow(i, _):
                wbuf[i, :] = ee2[b2, i, :] * 0.125 / (den2[b2, i, :] + 1e-16)
                return 0
            lax.fori_loop(0, CHUNK_B, wrow, 0)

            @pl.when(k > 0)
            def _():
                wait_scatter((b + 3) % 4, (b2 + 1) % 2)

            @pl.when(k + 3 < NCHUNKS_B)
            def _():
                fire_far(k + 3, (b + 3) % 4)

            @pl.when(k + 1 < NCHUNKS_B)
            def _():
                fire_near(k + 1, (b + 1) % 4, (b2 + 1) % 2)

            pltpu.make_async_copy(hs.at[idx4.at[b].at[0]], hs4.at[b],
                                  sem_h[b]).wait()

            def edge(e_, _):
                wrow_ = wbuf[e_, :]
                ws = [wrow_[h] for h in range(8)]
                acc = [None] * 4
                for h in range(8):
                    for j2 in range(2):
                        raw = hs4[b, e_, pl.ds(h * 64 + j2 * 32, 32)]
                        ti = plsc.bitcast(raw, jnp.int32)
                        fe = plsc.bitcast(ti << 16, jnp.float32)
                        fo = plsc.bitcast(ti & jnp.int32(-65536), jnp.float32)
                        te = ws[h] * fe
                        to = ws[h] * fo
                        qe, qo = 2 * j2, 2 * j2 + 1
                        acc[qe] = te if h == 0 else acc[qe] + te
                        acc[qo] = to if h == 0 else acc[qo] + to
                for q in range(4):
                    msg2[b2, e_, pl.ds(q * 16, 16)] = acc[q]
                return 0
            lax.fori_loop(0, CHUNK_B, edge, 0)
            pltpu.make_async_copy(msg2.at[b2], out_sh.at[idx4.at[b].at[1]],
                                  sem_s[b2]).start(add=True)

        fire_far(0, 0)
        fire_far(1, 1)
        fire_far(2, 2)
        fire_near(0, 0, 0)

        def quad(kk, _):
            for b in range(4):
                process(4 * kk + b, b, b % 2)
            return 0
        lax.fori_loop(0, NCHUNKS_B // 4, quad, 0)
        wait_scatter(3, 1)

        plsc.subcore_barrier()
        pltpu.sync_copy(out_sh.at[pl.ds(r0, ROWS_PER_TILE)],
                        op_o.at[cid, pl.ds(r0, ROWS_PER_TILE)])
        plsc.subcore_barrier()


def _tc2_body(opqa, opaq, bqa, baq, wo, bo, out):
    f1 = opaq[0] + opaq[1] + baq[...]
    f2 = opqa[0] + opqa[1] + bqa[...]
    out[...] = (jnp.dot(f1, wo[0:64, :], preferred_element_type=jnp.float32)
                + jnp.dot(f2, wo[64:128, :], preferred_element_type=jnp.float32)
                + bo[...])


def _full(shape):
    return pl.BlockSpec(shape, lambda i: (0,) * len(shape))


@jax.jit
def kernel(x_question, x_answer, edge_index_qa, edge_index_aq,
           W_src_qa, W_dst_qa, att_src_qa, att_dst_qa, bias_qa,
           W_src_aq, W_dst_aq, att_src_aq, att_dst_aq, bias_aq,
           W_out, b_out, ew_qa, ew_aq):
    f32 = jnp.float32
    eye = jnp.eye(HEADS, dtype=f32)

    def amat(att):
        a = (att[:, :, None] * eye[:, None, :]).reshape(HEADS * HID, HEADS)
        return jnp.pad(a, ((0, 0), (0, 16 - HEADS)))

    a_s_qa, a_d_qa = amat(att_src_qa), amat(att_dst_qa)
    a_s_aq, a_d_aq = amat(att_src_aq), amat(att_dst_aq)

    l16 = jnp.arange(16)
    perm = jnp.concatenate([2 * l16, 2 * l16 + 1, 32 + 2 * l16, 33 + 2 * l16])
    w_eff = jnp.concatenate([W_out[:64][perm], W_out[64:][perm]], axis=0)
    bias_qa_eff = bias_qa[perm]
    bias_aq_eff = bias_aq[perm]

    xq = jnp.pad(x_question, ((0, NP - N), (0, 0)))
    xa = jnp.pad(x_answer, ((0, NP - N), (0, 0)))

    pad_idx = jnp.full((EP - E,), PAD_NODE, dtype=jnp.int32)
    es_qa = jnp.concatenate([edge_index_qa[0].astype(jnp.int32), pad_idx])
    ed_qa = jnp.concatenate([edge_index_qa[1].astype(jnp.int32), pad_idx])
    es_aq = jnp.concatenate([edge_index_aq[0].astype(jnp.int32), pad_idx])
    ed_aq = jnp.concatenate([edge_index_aq[1].astype(jnp.int32), pad_idx])
    ep_qa = jnp.stack([es_qa.reshape(-1, CHUNK_B), ed_qa.reshape(-1, CHUNK_B)], 1)
    ep_aq = jnp.stack([es_aq.reshape(-1, CHUNK_B), ed_aq.reshape(-1, CHUNK_B)], 1)
    epa_qa = jnp.stack([es_qa.reshape(-1, CHUNK), ed_qa.reshape(-1, CHUNK)], 1)
    epa_aq = jnp.stack([es_aq.reshape(-1, CHUNK), ed_aq.reshape(-1, CHUNK)], 1)

    tc1 = pl.pallas_call(
        _tc1_body,
        grid=(GRID_N,),
        in_specs=[
            pl.BlockSpec((BN, D_IN), lambda i: (i, 0)),
            pl.BlockSpec((BN, D_IN), lambda i: (i, 0)),
            _full((D_IN, HEADS * HID)), _full((HEADS * HID, 16)),
            _full((D_IN, HEADS * HID)), _full((HEADS * HID, 16)),
            _full((D_IN, HEADS * HID)), _full((HEADS * HID, 16)),
            _full((D_IN, HEADS * HID)), _full((HEADS * HID, 16)),
        ],
        out_specs=[
            pl.BlockSpec((BN, HEADS * HID), lambda i: (i, 0)),
            pl.BlockSpec((BN, 16), lambda i: (i, 0)),
            pl.BlockSpec((BN, 16), lambda i: (i, 0)),
            pl.BlockSpec((BN, HEADS * HID), lambda i: (i, 0)),
            pl.BlockSpec((BN, 16), lambda i: (i, 0)),
            pl.BlockSpec((BN, 16), lambda i: (i, 0)),
        ],
        out_shape=[
            jax.ShapeDtypeStruct((NP, HEADS * HID), jnp.bfloat16),
            jax.ShapeDtypeStruct((NP, 16), f32),
            jax.ShapeDtypeStruct((NP, 16), f32),
            jax.ShapeDtypeStruct((NP, HEADS * HID), jnp.bfloat16),
            jax.ShapeDtypeStruct((NP, 16), f32),
            jax.ShapeDtypeStruct((NP, 16), f32),
        ],
    )
    hs_qa, al_s_qa, al_d_qa, hs_aq, al_s_aq, al_d_aq = tc1(
        xq, xa, W_src_qa, a_s_qa, W_dst_qa, a_d_qa,
        W_src_aq, a_s_aq, W_dst_aq, a_d_aq)

    mesh = plsc.VectorSubcoreMesh(core_axis_name="c", subcore_axis_name="s")
    z16 = jnp.zeros((NP, 16), f32)
    z64 = jnp.zeros((NP, HID), f32)

    phase_a = pl.kernel(
        _sc_phase_a_body,
        out_type=[
            jax.ShapeDtypeStruct((EP, 16), f32),
            jax.ShapeDtypeStruct((EP, 16), f32),
            jax.ShapeDtypeStruct((2, NP, 16), f32),
            jax.ShapeDtypeStruct((2, NP, 16), f32),
        ],
        mesh=mesh,
        compiler_params=pltpu.CompilerParams(use_tc_tiling_on_sc=False),
        scratch_types=[
            pltpu.VMEM((4, 2, CHUNK), jnp.int32),
            pltpu.VMEM((4, CHUNK, 16), f32),
            pltpu.VMEM((4, CHUNK, 16), f32),
            pltpu.VMEM((4, CHUNK, 16), f32),
            pltpu.VMEM_SHARED((NP, 16), f32),
        ] + [pltpu.SemaphoreType.DMA] * 16,
    )
    ee_qa, ee_aq, dp_qa, dp_aq = phase_a(
        epa_qa, epa_aq, al_s_qa, al_d_qa, al_s_aq, al_d_aq, z16)

    den_qa = dp_qa[0] + dp_qa[1]
    den_aq = dp_aq[0] + dp_aq[1]

    phase_b = pl.kernel(
        _sc_phase_b_body,
        out_type=[
            jax.ShapeDtypeStruct((2, NP, HID), f32),
            jax.ShapeDtypeStruct((2, NP, HID), f32),
        ],
        mesh=mesh,
        compiler_params=pltpu.CompilerParams(use_tc_tiling_on_sc=False,
                                             needs_layout_passes=False),
        scratch_types=[
            pltpu.VMEM((4, 2, CHUNK_B), jnp.int32),
            pltpu.VMEM((2, CHUNK_B, 16), f32),
            pltpu.VMEM((2, CHUNK_B, 16), f32),
            pltpu.VMEM((CHUNK_B, 16), f32),
            pltpu.VMEM((4, CHUNK_B, HEADS * HID), jnp.bfloat16),
            pltpu.VMEM((2, CHUNK_B, HID), f32),
            pltpu.VMEM_SHARED((NP, HID), f32),
        ] + [pltpu.SemaphoreType.DMA] * 10,
    )
    op_qa, op_aq = phase_b(
        ep_qa, ep_aq, ee_qa, ee_aq, den_qa, den_aq,
        hs_qa, hs_aq, z64)

    tc2 = pl.pallas_call(
        _tc2_body,
        grid=(GRID_N,),
        in_specs=[
            pl.BlockSpec((2, BN, HID), lambda i: (0, i, 0)),
            pl.BlockSpec((2, BN, HID), lambda i: (0, i, 0)),
            _full((1, HID)), _full((1, HID)),
            _full((2 * HID, NC_OUT)), _full((1, NC_OUT)),
        ],
        out_specs=pl.BlockSpec((BN, NC_OUT), lambda i: (i, 0)),
        out_shape=jax.ShapeDtypeStruct((NP, NC_OUT), f32),
    )
    preds = tc2(op_qa, op_aq, bias_qa_eff.reshape(1, HID),
                bias_aq_eff.reshape(1, HID), w_eff, b_out.reshape(1, NC_OUT))
    return (preds[:N], ew_qa, ew_aq)


# speedup vs baseline: 1.0678x; 1.0678x over previous
import functools
import jax
import jax.numpy as jnp
from jax import lax
from jax.experimental import pallas as pl
from jax.experimental.pallas import tpu as pltpu, tpu_sc as plsc

N = 10000
E = 160000
D_IN = 128
HID = 64
HEADS = 8
NC_OUT = 4

NP = 10240
EP = 163840
PAD_NODE = N

NUM_TILES = 32
EDGES_PER_TILE = EP // NUM_TILES
CHUNK = 128
NCHUNKS = EDGES_PER_TILE // CHUNK
CHUNK_B = 64
NCHUNKS_B = EDGES_PER_TILE // CHUNK_B
ROWS_PER_TILE = NP // 16

BN = 256
GRID_N = NP // BN


def _tc1_body(xq, xa, wsqa, asqa, wdqa, adqa, wsaq, asaq, wdaq, adaq,
              hs_qa, al_s_qa, al_d_qa, hs_aq, al_s_aq, al_d_aq):
    xqb = xq[...]
    xab = xa[...]
    hq = jnp.dot(xqb, wsqa[...], preferred_element_type=jnp.float32)
    hs_qa[...] = hq.astype(jnp.bfloat16)
    al_s_qa[...] = jnp.dot(hq, asqa[...], preferred_element_type=jnp.float32)
    vd_qa = jnp.dot(wdqa[...], adqa[...], preferred_element_type=jnp.float32)
    al_d_qa[...] = jnp.dot(xab, vd_qa, preferred_element_type=jnp.float32)
    ha = jnp.dot(xab, wsaq[...], preferred_element_type=jnp.float32)
    hs_aq[...] = ha.astype(jnp.bfloat16)
    al_s_aq[...] = jnp.dot(ha, asaq[...], preferred_element_type=jnp.float32)
    vd_aq = jnp.dot(wdaq[...], adaq[...], preferred_element_type=jnp.float32)
    al_d_aq[...] = jnp.dot(xqb, vd_aq, preferred_element_type=jnp.float32)


def _sc_phase_a_body(epa_qa, epa_aq, asq, adq, asa, ada, z16,
                     ee_qa, ee_aq, dp_qa, dp_aq,
                     idx4, aa4, bb4, ee4, den_sh,
                     sa0, sa1, sa2, sa3, sb0, sb1, sb2, sb3,
                     sw0, sw1, sw2, sw3, ss0, ss1, ss2, ss3):
    cid = lax.axis_index("c")
    sid = lax.axis_index("s")
    wid = cid * 16 + sid
    r0 = sid * ROWS_PER_TILE
    sem_a = (sa0, sa1, sa2, sa3)
    sem_b = (sb0, sb1, sb2, sb3)
    sem_w = (sw0, sw1, sw2, sw3)
    sem_s = (ss0, ss1, ss2, ss3)

    for epa, asrc, adst, ee_o, dp_o in (
        (epa_qa, asq, adq, ee_qa, dp_qa),
        (epa_aq, asa, ada, ee_aq, dp_aq),
    ):
        pltpu.sync_copy(z16.at[pl.ds(r0, ROWS_PER_TILE)],
                        den_sh.at[pl.ds(r0, ROWS_PER_TILE)])
        plsc.subcore_barrier()

        def fire(k, b, epa=epa, asrc=asrc, adst=adst):
            g = wid * NCHUNKS + k
            pltpu.sync_copy(epa.at[g], idx4.at[b])
            pltpu.make_async_copy(asrc.at[idx4.at[b].at[0]], aa4.at[b],
                                  sem_a[b]).start()
            pltpu.make_async_copy(adst.at[idx4.at[b].at[1]], bb4.at[b],
                                  sem_b[b]).start()

        def drain(k, b, ee_o=ee_o):
            base = (wid * NCHUNKS + k) * CHUNK
            pltpu.make_async_copy(ee4.at[b], ee_o.at[pl.ds(base, CHUNK)],
                                  sem_w[b]).wait()
            pltpu.make_async_copy(ee4.at[b], den_sh.at[idx4.at[b].at[1]],
                                  sem_s[b]).wait()

        def process(k, b, asrc=asrc, adst=adst, ee_o=ee_o):
            base = (wid * NCHUNKS + k) * CHUNK
            pltpu.make_async_copy(asrc.at[idx4.at[b].at[0]], aa4.at[b],
                                  sem_a[b]).wait()
            pltpu.make_async_copy(adst.at[idx4.at[b].at[1]], bb4.at[b],
                                  sem_b[b]).wait()

            @pl.when(k > 0)
            def _():
                drain(k - 1, (b + 3) % 4)

            @pl.when(k + 3 < NCHUNKS)
            def _():
                fire(k + 3, (b + 3) % 4)

            def row(i, _):
                sv = aa4[b, i, :] + bb4[b, i, :]
                e = jnp.maximum(sv, 0.2 * sv)
                ee4[b, i, :] = jnp.exp(e)
                return 0
            lax.fori_loop(0, CHUNK, row, 0)
            pltpu.make_async_copy(ee4.at[b], ee_o.at[pl.ds(base, CHUNK)],
                                  sem_w[b]).start()
            pltpu.make_async_copy(ee4.at[b], den_sh.at[idx4.at[b].at[1]],
                                  sem_s[b]).start(add=True)

        fire(0, 0)
        fire(1, 1)
        fire(2, 2)

        def quad(kk, _):
            for b in range(4):
                process(4 * kk + b, b)
            return 0
        lax.fori_loop(0, NCHUNKS // 4, quad, 0)
        drain(NCHUNKS - 1, 3)

        plsc.subcore_barrier()
        pltpu.sync_copy(den_sh.at[pl.ds(r0, ROWS_PER_TILE)],
                        dp_o.at[cid, pl.ds(r0, ROWS_PER_TILE)])
        plsc.subcore_barrier()


def _sc_phase_b_body(ep_qa, ep_aq, ee_qa, ee_aq, den_qa, den_aq,
                     hs_qa, hs_aq, z64,
                     op_qa, op_aq,
                     idx4, ee2, den2, wbuf, hs2, msg2, out_sh,
                     se0, se1, sd0, sd1, sh0, sh1, ss0, ss1):
    cid = lax.axis_index("c")
    sid = lax.axis_index("s")
    wid = cid * 16 + sid
    r0 = sid * ROWS_PER_TILE
    sem_e = (se0, se1)
    sem_d = (sd0, sd1)
    sem_h = (sh0, sh1)
    sem_s = (ss0, ss1)

    for ep, ee, den, hs, op_o in (
        (ep_qa, ee_qa, den_qa, hs_qa, op_qa),
        (ep_aq, ee_aq, den_aq, hs_aq, op_aq),
    ):
        pltpu.sync_copy(z64.at[pl.ds(r0, ROWS_PER_TILE)],
                        out_sh.at[pl.ds(r0, ROWS_PER_TILE)])
        plsc.subcore_barrier()

        def fire_idx(k, b, ep=ep):
            g = wid * NCHUNKS_B + k
            pltpu.sync_copy(ep.at[g], idx4.at[b])

        def fire_hs(k, b, bh, hs=hs):
            pltpu.make_async_copy(hs.at[idx4.at[b].at[0]], hs2.at[bh],
                                  sem_h[bh]).start()

        def fire_near(k, b, b2, ee=ee, den=den):
            base = (wid * NCHUNKS_B + k) * CHUNK_B
            pltpu.make_async_copy(den.at[idx4.at[b].at[1]], den2.at[b2],
                                  sem_d[b2]).start()
            pltpu.make_async_copy(ee.at[pl.ds(base, CHUNK_B)], ee2.at[b2],
                                  sem_e[b2]).start()

        def wait_scatter(b, b2):
            pltpu.make_async_copy(msg2.at[b2], out_sh.at[idx4.at[b].at[1]],
                                  sem_s[b2]).wait()

        def process(k, b, b2, ee=ee, den=den, hs=hs):
            pltpu.make_async_copy(den.at[idx4.at[b].at[1]], den2.at[b2],
                                  sem_d[b2]).wait()
            pltpu.make_async_copy(ee.at[pl.ds(0, CHUNK_B)], ee2.at[b2],
                                  sem_e[b2]).wait()

            def wrow(i, _):
                wbuf[i, :] = ee2[b2, i, :] * 0.125 / (den2[b2, i, :] + 1e-16)
                return 0
            lax.fori_loop(0, CHUNK_B, wrow, 0)

            @pl.when(k > 0)
            def _():
                wait_scatter((b + 3) % 4, (b2 + 1) % 2)

            @pl.when(k + 3 < NCHUNKS_B)
            def _():
                fire_idx(k + 3, (b + 3) % 4)

            @pl.when(k + 1 < NCHUNKS_B)
            def _():
                fire_hs(k + 1, (b + 1) % 4, (b2 + 1) % 2)
                fire_near(k + 1, (b + 1) % 4, (b2 + 1) % 2)

            pltpu.make_async_copy(hs.at[idx4.at[b].at[0]], hs2.at[b2],
                                  sem_h[b2]).wait()

            def edge(e_, _):
                wrow_ = wbuf[e_, :]
                ws = [wrow_[h] for h in range(8)]
                acc = [None] * 4
                for h in range(8):
                    for j2 in range(2):
                        raw = hs2[b2, e_, pl.ds(h * 64 + j2 * 32, 32)]
                        ti = plsc.bitcast(raw, jnp.int32)
                        fe = plsc.bitcast(ti << 16, jnp.float32)
                        fo = plsc.bitcast(ti & jnp.int32(-65536), jnp.float32)
                        te = ws[h] * fe
                        to = ws[h] * fo
                        qe, qo = 2 * j2, 2 * j2 + 1
                        acc[qe] = te if h == 0 else acc[qe] + te
                        acc[qo] = to if h == 0 else acc[qo] + to
                for q in range(4):
                    msg2[b2, e_, pl.ds(q * 16, 16)] = acc[q]
                return 0
            lax.fori_loop(0, CHUNK_B, edge, 0)
            pltpu.make_async_copy(msg2.at[b2], out_sh.at[idx4.at[b].at[1]],
                                  sem_s[b2]).start(add=True)

        fire_idx(0, 0)
        fire_idx(1, 1)
        fire_idx(2, 2)
        fire_hs(0, 0, 0)
        fire_near(0, 0, 0)

        def quad(kk, _):
            for b in range(4):
                process(4 * kk + b, b, b % 2)
            return 0
        lax.fori_loop(0, NCHUNKS_B // 4, quad, 0)
        wait_scatter(3, 1)

        plsc.subcore_barrier()
        pltpu.sync_copy(out_sh.at[pl.ds(r0, ROWS_PER_TILE)],
                        op_o.at[cid, pl.ds(r0, ROWS_PER_TILE)])
        plsc.subcore_barrier()


def _tc2_body(opqa, opaq, bqa, baq, wo, bo, out):
    f1 = opaq[0] + opaq[1] + baq[...]
    f2 = opqa[0] + opqa[1] + bqa[...]
    out[...] = (jnp.dot(f1, wo[0:64, :], preferred_element_type=jnp.float32)
                + jnp.dot(f2, wo[64:128, :], preferred_element_type=jnp.float32)
                + bo[...])


def _full(shape):
    return pl.BlockSpec(shape, lambda i: (0,) * len(shape))


@jax.jit
def kernel(x_question, x_answer, edge_index_qa, edge_index_aq,
           W_src_qa, W_dst_qa, att_src_qa, att_dst_qa, bias_qa,
           W_src_aq, W_dst_aq, att_src_aq, att_dst_aq, bias_aq,
           W_out, b_out, ew_qa, ew_aq):
    f32 = jnp.float32
    eye = jnp.eye(HEADS, dtype=f32)

    def amat(att):
        a = (att[:, :, None] * eye[:, None, :]).reshape(HEADS * HID, HEADS)
        return jnp.pad(a, ((0, 0), (0, 16 - HEADS)))

    a_s_qa, a_d_qa = amat(att_src_qa), amat(att_dst_qa)
    a_s_aq, a_d_aq = amat(att_src_aq), amat(att_dst_aq)

    l16 = jnp.arange(16)
    perm = jnp.concatenate([2 * l16, 2 * l16 + 1, 32 + 2 * l16, 33 + 2 * l16])
    w_eff = jnp.concatenate([W_out[:64][perm], W_out[64:][perm]], axis=0)
    bias_qa_eff = bias_qa[perm]
    bias_aq_eff = bias_aq[perm]

    xq = jnp.pad(x_question, ((0, NP - N), (0, 0)))
    xa = jnp.pad(x_answer, ((0, NP - N), (0, 0)))

    pad_idx = jnp.full((EP - E,), PAD_NODE, dtype=jnp.int32)
    es_qa = jnp.concatenate([edge_index_qa[0].astype(jnp.int32), pad_idx])
    ed_qa = jnp.concatenate([edge_index_qa[1].astype(jnp.int32), pad_idx])
    es_aq = jnp.concatenate([edge_index_aq[0].astype(jnp.int32), pad_idx])
    ed_aq = jnp.concatenate([edge_index_aq[1].astype(jnp.int32), pad_idx])
    ep_qa = jnp.stack([es_qa.reshape(-1, CHUNK_B), ed_qa.reshape(-1, CHUNK_B)], 1)
    ep_aq = jnp.stack([es_aq.reshape(-1, CHUNK_B), ed_aq.reshape(-1, CHUNK_B)], 1)
    epa_qa = jnp.stack([es_qa.reshape(-1, CHUNK), ed_qa.reshape(-1, CHUNK)], 1)
    epa_aq = jnp.stack([es_aq.reshape(-1, CHUNK), ed_aq.reshape(-1, CHUNK)], 1)

    tc1 = pl.pallas_call(
        _tc1_body,
        grid=(GRID_N,),
        in_specs=[
            pl.BlockSpec((BN, D_IN), lambda i: (i, 0)),
            pl.BlockSpec((BN, D_IN), lambda i: (i, 0)),
            _full((D_IN, HEADS * HID)), _full((HEADS * HID, 16)),
            _full((D_IN, HEADS * HID)), _full((HEADS * HID, 16)),
            _full((D_IN, HEADS * HID)), _full((HEADS * HID, 16)),
            _full((D_IN, HEADS * HID)), _full((HEADS * HID, 16)),
        ],
        out_specs=[
            pl.BlockSpec((BN, HEADS * HID), lambda i: (i, 0)),
            pl.BlockSpec((BN, 16), lambda i: (i, 0)),
            pl.BlockSpec((BN, 16), lambda i: (i, 0)),
            pl.BlockSpec((BN, HEADS * HID), lambda i: (i, 0)),
            pl.BlockSpec((BN, 16), lambda i: (i, 0)),
            pl.BlockSpec((BN, 16), lambda i: (i, 0)),
        ],
        out_shape=[
            jax.ShapeDtypeStruct((NP, HEADS * HID), jnp.bfloat16),
            jax.ShapeDtypeStruct((NP, 16), f32),
            jax.ShapeDtypeStruct((NP, 16), f32),
            jax.ShapeDtypeStruct((NP, HEADS * HID), jnp.bfloat16),
            jax.ShapeDtypeStruct((NP, 16), f32),
            jax.ShapeDtypeStruct((NP, 16), f32),
        ],
    )
    hs_qa, al_s_qa, al_d_qa, hs_aq, al_s_aq, al_d_aq = tc1(
        xq, xa, W_src_qa, a_s_qa, W_dst_qa, a_d_qa,
        W_src_aq, a_s_aq, W_dst_aq, a_d_aq)

    mesh = plsc.VectorSubcoreMesh(core_axis_name="c", subcore_axis_name="s")
    z16 = jnp.zeros((NP, 16), f32)
    z64 = jnp.zeros((NP, HID), f32)

    phase_a = pl.kernel(
        _sc_phase_a_body,
        out_type=[
            jax.ShapeDtypeStruct((EP, 16), f32),
            jax.ShapeDtypeStruct((EP, 16), f32),
            jax.ShapeDtypeStruct((2, NP, 16), f32),
            jax.ShapeDtypeStruct((2, NP, 16), f32),
        ],
        mesh=mesh,
        compiler_params=pltpu.CompilerParams(use_tc_tiling_on_sc=False),
        scratch_types=[
            pltpu.VMEM((4, 2, CHUNK), jnp.int32),
            pltpu.VMEM((4, CHUNK, 16), f32),
            pltpu.VMEM((4, CHUNK, 16), f32),
            pltpu.VMEM((4, CHUNK, 16), f32),
            pltpu.VMEM_SHARED((NP, 16), f32),
        ] + [pltpu.SemaphoreType.DMA] * 16,
    )
    ee_qa, ee_aq, dp_qa, dp_aq = phase_a(
        epa_qa, epa_aq, al_s_qa, al_d_qa, al_s_aq, al_d_aq, z16)

    den_qa = dp_qa[0] + dp_qa[1]
    den_aq = dp_aq[0] + dp_aq[1]

    phase_b = pl.kernel(
        _sc_phase_b_body,
        out_type=[
            jax.ShapeDtypeStruct((2, NP, HID), f32),
            jax.ShapeDtypeStruct((2, NP, HID), f32),
        ],
        mesh=mesh,
        compiler_params=pltpu.CompilerParams(use_tc_tiling_on_sc=False,
                                             needs_layout_passes=False),
        scratch_types=[
            pltpu.VMEM((4, 2, CHUNK_B), jnp.int32),
            pltpu.VMEM((2, CHUNK_B, 16), f32),
            pltpu.VMEM((2, CHUNK_B, 16), f32),
            pltpu.VMEM((CHUNK_B, 16), f32),
            pltpu.VMEM((2, CHUNK_B, HEADS * HID), jnp.bfloat16),
            pltpu.VMEM((2, CHUNK_B, HID), f32),
            pltpu.VMEM_SHARED((NP, HID), f32),
        ] + [pltpu.SemaphoreType.DMA] * 8,
    )
    op_qa, op_aq = phase_b(
        ep_qa, ep_aq, ee_qa, ee_aq, den_qa, den_aq,
        hs_qa, hs_aq, z64)

    tc2 = pl.pallas_call(
        _tc2_body,
        grid=(GRID_N,),
        in_specs=[
            pl.BlockSpec((2, BN, HID), lambda i: (0, i, 0)),
            pl.BlockSpec((2, BN, HID), lambda i: (0, i, 0)),
            _full((1, HID)), _full((1, HID)),
            _full((2 * HID, NC_OUT)), _full((1, NC_OUT)),
        ],
        out_specs=pl.BlockSpec((BN, NC_OUT), lambda i: (i, 0)),
        out_shape=jax.ShapeDtypeStruct((NP, NC_OUT), f32),
    )
    preds = tc2(op_qa, op_aq, bias_qa_eff.reshape(1, HID),
                bias_aq_eff.reshape(1, HID), w_eff, b_out.reshape(1, NC_OUT))
    return (preds[:N], ew_qa, ew_aq)


# speedup vs baseline: 1.0702x; 1.0023x over previous
import functools
import jax
import jax.numpy as jnp
from jax import lax
from jax.experimental import pallas as pl
from jax.experimental.pallas import tpu as pltpu, tpu_sc as plsc

N = 10000
E = 160000
D_IN = 128
HID = 64
HEADS = 8
NC_OUT = 4

NP = 10240
EP = 163840
PAD_NODE = N

NUM_TILES = 32
EDGES_PER_TILE = EP // NUM_TILES
CHUNK = 128
NCHUNKS = EDGES_PER_TILE // CHUNK
CHUNK_B = 80
NCHUNKS_B = EDGES_PER_TILE // CHUNK_B
ROWS_PER_TILE = NP // 16

BN = 256
GRID_N = NP // BN


def _tc1_body(xq, xa, wsqa, asqa, wdqa, adqa, wsaq, asaq, wdaq, adaq,
              hs_qa, al_s_qa, al_d_qa, hs_aq, al_s_aq, al_d_aq):
    xqb = xq[...]
    xab = xa[...]
    hq = jnp.dot(xqb, wsqa[...], preferred_element_type=jnp.float32)
    hs_qa[...] = hq.astype(jnp.bfloat16)
    al_s_qa[...] = jnp.dot(hq, asqa[...], preferred_element_type=jnp.float32)
    vd_qa = jnp.dot(wdqa[...], adqa[...], preferred_element_type=jnp.float32)
    al_d_qa[...] = jnp.dot(xab, vd_qa, preferred_element_type=jnp.float32)
    ha = jnp.dot(xab, wsaq[...], preferred_element_type=jnp.float32)
    hs_aq[...] = ha.astype(jnp.bfloat16)
    al_s_aq[...] = jnp.dot(ha, asaq[...], preferred_element_type=jnp.float32)
    vd_aq = jnp.dot(wdaq[...], adaq[...], preferred_element_type=jnp.float32)
    al_d_aq[...] = jnp.dot(xqb, vd_aq, preferred_element_type=jnp.float32)


def _sc_phase_a_body(epa_qa, epa_aq, asq, adq, asa, ada, z16,
                     ee_qa, ee_aq, dp_qa, dp_aq,
                     idx4, aa4, bb4, ee4, den_sh,
                     sa0, sa1, sa2, sa3, sb0, sb1, sb2, sb3,
                     sw0, sw1, sw2, sw3, ss0, ss1, ss2, ss3):
    cid = lax.axis_index("c")
    sid = lax.axis_index("s")
    wid = cid * 16 + sid
    r0 = sid * ROWS_PER_TILE
    sem_a = (sa0, sa1, sa2, sa3)
    sem_b = (sb0, sb1, sb2, sb3)
    sem_w = (sw0, sw1, sw2, sw3)
    sem_s = (ss0, ss1, ss2, ss3)

    for epa, asrc, adst, ee_o, dp_o in (
        (epa_qa, asq, adq, ee_qa, dp_qa),
        (epa_aq, asa, ada, ee_aq, dp_aq),
    ):
        pltpu.sync_copy(z16.at[pl.ds(r0, ROWS_PER_TILE)],
                        den_sh.at[pl.ds(r0, ROWS_PER_TILE)])
        plsc.subcore_barrier()

        def fire(k, b, epa=epa, asrc=asrc, adst=adst):
            g = wid * NCHUNKS + k
            pltpu.sync_copy(epa.at[g], idx4.at[b])
            pltpu.make_async_copy(asrc.at[idx4.at[b].at[0]], aa4.at[b],
                                  sem_a[b]).start()
            pltpu.make_async_copy(adst.at[idx4.at[b].at[1]], bb4.at[b],
                                  sem_b[b]).start()

        def drain(k, b, ee_o=ee_o):
            base = (wid * NCHUNKS + k) * CHUNK
            pltpu.make_async_copy(ee4.at[b], ee_o.at[pl.ds(base, CHUNK)],
                                  sem_w[b]).wait()
            pltpu.make_async_copy(ee4.at[b], den_sh.at[idx4.at[b].at[1]],
                                  sem_s[b]).wait()

        def process(k, b, asrc=asrc, adst=adst, ee_o=ee_o):
            base = (wid * NCHUNKS + k) * CHUNK
            pltpu.make_async_copy(asrc.at[idx4.at[b].at[0]], aa4.at[b],
                                  sem_a[b]).wait()
            pltpu.make_async_copy(adst.at[idx4.at[b].at[1]], bb4.at[b],
                                  sem_b[b]).wait()

            @pl.when(k > 0)
            def _():
                drain(k - 1, (b + 3) % 4)

            @pl.when(k + 3 < NCHUNKS)
            def _():
                fire(k + 3, (b + 3) % 4)

            def row(i, _):
                sv = aa4[b, i, :] + bb4[b, i, :]
                e = jnp.maximum(sv, 0.2 * sv)
                ee4[b, i, :] = jnp.exp(e)
                return 0
            lax.fori_loop(0, CHUNK, row, 0)
            pltpu.make_async_copy(ee4.at[b], ee_o.at[pl.ds(base, CHUNK)],
                                  sem_w[b]).start()
            pltpu.make_async_copy(ee4.at[b], den_sh.at[idx4.at[b].at[1]],
                                  sem_s[b]).start(add=True)

        fire(0, 0)
        fire(1, 1)
        fire(2, 2)

        def quad(kk, _):
            for b in range(4):
                process(4 * kk + b, b)
            return 0
        lax.fori_loop(0, NCHUNKS // 4, quad, 0)
        drain(NCHUNKS - 1, 3)

        plsc.subcore_barrier()
        pltpu.sync_copy(den_sh.at[pl.ds(r0, ROWS_PER_TILE)],
                        dp_o.at[cid, pl.ds(r0, ROWS_PER_TILE)])
        plsc.subcore_barrier()


def _sc_phase_b_body(ep_qa, ep_aq, ee_qa, ee_aq, den_qa, den_aq,
                     hs_qa, hs_aq, z64,
                     op_qa, op_aq,
                     idx4, ee2, den2, wbuf, hs2, msg2, out_sh,
                     se0, se1, sd0, sd1, sh0, sh1, ss0, ss1):
    cid = lax.axis_index("c")
    sid = lax.axis_index("s")
    wid = cid * 16 + sid
    r0 = sid * ROWS_PER_TILE
    sem_e = (se0, se1)
    sem_d = (sd0, sd1)
    sem_h = (sh0, sh1)
    sem_s = (ss0, ss1)

    for ep, ee, den, hs, op_o in (
        (ep_qa, ee_qa, den_qa, hs_qa, op_qa),
        (ep_aq, ee_aq, den_aq, hs_aq, op_aq),
    ):
        pltpu.sync_copy(z64.at[pl.ds(r0, ROWS_PER_TILE)],
                        out_sh.at[pl.ds(r0, ROWS_PER_TILE)])
        plsc.subcore_barrier()

        def fire_idx(k, b, ep=ep):
            g = wid * NCHUNKS_B + k
            pltpu.sync_copy(ep.at[g], idx4.at[b])

        def fire_hs(k, b, bh, hs=hs):
            pltpu.make_async_copy(hs.at[idx4.at[b].at[0]], hs2.at[bh],
                                  sem_h[bh]).start()

        def fire_near(k, b, b2, ee=ee, den=den):
            base = (wid * NCHUNKS_B + k) * CHUNK_B
            pltpu.make_async_copy(den.at[idx4.at[b].at[1]], den2.at[b2],
                                  sem_d[b2]).start()
            pltpu.make_async_copy(ee.at[pl.ds(base, CHUNK_B)], ee2.at[b2],
                                  sem_e[b2]).start()

        def wait_scatter(b, b2):
            pltpu.make_async_copy(msg2.at[b2], out_sh.at[idx4.at[b].at[1]],
                                  sem_s[b2]).wait()

        def process(k, b, b2, ee=ee, den=den, hs=hs):
            pltpu.make_async_copy(den.at[idx4.at[b].at[1]], den2.at[b2],
                                  sem_d[b2]).wait()
            pltpu.make_async_copy(ee.at[pl.ds(0, CHUNK_B)], ee2.at[b2],
                                  sem_e[b2]).wait()

            def wrow(i, _):
                wbuf[i, :] = ee2[b2, i, :] * 0.125 / (den2[b2, i, :] + 1e-16)
                return 0
            lax.fori_loop(0, CHUNK_B, wrow, 0)

            @pl.when(k > 0)
            def _():
                wait_scatter((b + 3) % 4, (b2 + 1) % 2)

            @pl.when(k + 3 < NCHUNKS_B)
            def _():
                fire_idx(k + 3, (b + 3) % 4)

            @pl.when(k + 1 < NCHUNKS_B)
            def _():
                fire_hs(k + 1, (b + 1) % 4, (b2 + 1) % 2)
                fire_near(k + 1, (b + 1) % 4, (b2 + 1) % 2)

            pltpu.make_async_copy(hs.at[idx4.at[b].at[0]], hs2.at[b2],
                                  sem_h[b2]).wait()

            def edge(e_, _):
                wrow_ = wbuf[e_, :]
                ws = [wrow_[h] for h in range(8)]
                acc = [None] * 4
                for h in range(8):
                    for j2 in range(2):
                        raw = hs2[b2, e_, pl.ds(h * 64 + j2 * 32, 32)]
                        ti = plsc.bitcast(raw, jnp.int32)
                        fe = plsc.bitcast(ti << 16, jnp.float32)
                        fo = plsc.bitcast(ti & jnp.int32(-65536), jnp.float32)
                        te = ws[h] * fe
                        to = ws[h] * fo
                        qe, qo = 2 * j2, 2 * j2 + 1
                        acc[qe] = te if h == 0 else acc[qe] + te
                        acc[qo] = to if h == 0 else acc[qo] + to
                for q in range(4):
                    msg2[b2, e_, pl.ds(q * 16, 16)] = acc[q]
                return 0
            lax.fori_loop(0, CHUNK_B, edge, 0)
            pltpu.make_async_copy(msg2.at[b2], out_sh.at[idx4.at[b].at[1]],
                                  sem_s[b2]).start(add=True)

        fire_idx(0, 0)
        fire_idx(1, 1)
        fire_idx(2, 2)
        fire_hs(0, 0, 0)
        fire_near(0, 0, 0)

        def quad(kk, _):
            for b in range(4):
                process(4 * kk + b, b, b % 2)
            return 0
        lax.fori_loop(0, NCHUNKS_B // 4, quad, 0)
        wait_scatter(3, 1)

        plsc.subcore_barrier()
        pltpu.sync_copy(out_sh.at[pl.ds(r0, ROWS_PER_TILE)],
                        op_o.at[cid, pl.ds(r0, ROWS_PER_TILE)])
        plsc.subcore_barrier()


def _tc2_body(opqa, opaq, bqa, baq, wo, bo, out):
    f1 = opaq[0] + opaq[1] + baq[...]
    f2 = opqa[0] + opqa[1] + bqa[...]
    out[...] = (jnp.dot(f1, wo[0:64, :], preferred_element_type=jnp.float32)
                + jnp.dot(f2, wo[64:128, :], preferred_element_type=jnp.float32)
                + bo[...])


def _full(shape):
    return pl.BlockSpec(shape, lambda i: (0,) * len(shape))


@jax.jit
def kernel(x_question, x_answer, edge_index_qa, edge_index_aq,
           W_src_qa, W_dst_qa, att_src_qa, att_dst_qa, bias_qa,
           W_src_aq, W_dst_aq, att_src_aq, att_dst_aq, bias_aq,
           W_out, b_out, ew_qa, ew_aq):
    f32 = jnp.float32
    eye = jnp.eye(HEADS, dtype=f32)

    def amat(att):
        a = (att[:, :, None] * eye[:, None, :]).reshape(HEADS * HID, HEADS)
        return jnp.pad(a, ((0, 0), (0, 16 - HEADS)))

    a_s_qa, a_d_qa = amat(att_src_qa), amat(att_dst_qa)
    a_s_aq, a_d_aq = amat(att_src_aq), amat(att_dst_aq)

    l16 = jnp.arange(16)
    perm = jnp.concatenate([2 * l16, 2 * l16 + 1, 32 + 2 * l16, 33 + 2 * l16])
    w_eff = jnp.concatenate([W_out[:64][perm], W_out[64:][perm]], axis=0)
    bias_qa_eff = bias_qa[perm]
    bias_aq_eff = bias_aq[perm]

    xq = jnp.pad(x_question, ((0, NP - N), (0, 0)))
    xa = jnp.pad(x_answer, ((0, NP - N), (0, 0)))

    pad_idx = jnp.full((EP - E,), PAD_NODE, dtype=jnp.int32)
    es_qa = jnp.concatenate([edge_index_qa[0].astype(jnp.int32), pad_idx])
    ed_qa = jnp.concatenate([edge_index_qa[1].astype(jnp.int32), pad_idx])
    es_aq = jnp.concatenate([edge_index_aq[0].astype(jnp.int32), pad_idx])
    ed_aq = jnp.concatenate([edge_index_aq[1].astype(jnp.int32), pad_idx])
    ep_qa = jnp.stack([es_qa.reshape(-1, CHUNK_B), ed_qa.reshape(-1, CHUNK_B)], 1)
    ep_aq = jnp.stack([es_aq.reshape(-1, CHUNK_B), ed_aq.reshape(-1, CHUNK_B)], 1)
    epa_qa = jnp.stack([es_qa.reshape(-1, CHUNK), ed_qa.reshape(-1, CHUNK)], 1)
    epa_aq = jnp.stack([es_aq.reshape(-1, CHUNK), ed_aq.reshape(-1, CHUNK)], 1)

    tc1 = pl.pallas_call(
        _tc1_body,
        grid=(GRID_N,),
        in_specs=[
            pl.BlockSpec((BN, D_IN), lambda i: (i, 0)),
            pl.BlockSpec((BN, D_IN), lambda i: (i, 0)),
            _full((D_IN, HEADS * HID)), _full((HEADS * HID, 16)),
            _full((D_IN, HEADS * HID)), _full((HEADS * HID, 16)),
            _full((D_IN, HEADS * HID)), _full((HEADS * HID, 16)),
            _full((D_IN, HEADS * HID)), _full((HEADS * HID, 16)),
        ],
        out_specs=[
            pl.BlockSpec((BN, HEADS * HID), lambda i: (i, 0)),
            pl.BlockSpec((BN, 16), lambda i: (i, 0)),
            pl.BlockSpec((BN, 16), lambda i: (i, 0)),
            pl.BlockSpec((BN, HEADS * HID), lambda i: (i, 0)),
            pl.BlockSpec((BN, 16), lambda i: (i, 0)),
            pl.BlockSpec((BN, 16), lambda i: (i, 0)),
        ],
        out_shape=[
            jax.ShapeDtypeStruct((NP, HEADS * HID), jnp.bfloat16),
            jax.ShapeDtypeStruct((NP, 16), f32),
            jax.ShapeDtypeStruct((NP, 16), f32),
            jax.ShapeDtypeStruct((NP, HEADS * HID), jnp.bfloat16),
            jax.ShapeDtypeStruct((NP, 16), f32),
            jax.ShapeDtypeStruct((NP, 16), f32),
        ],
    )
    hs_qa, al_s_qa, al_d_qa, hs_aq, al_s_aq, al_d_aq = tc1(
        xq, xa, W_src_qa, a_s_qa, W_dst_qa, a_d_qa,
        W_src_aq, a_s_aq, W_dst_aq, a_d_aq)

    mesh = plsc.VectorSubcoreMesh(core_axis_name="c", subcore_axis_name="s")
    z16 = jnp.zeros((NP, 16), f32)
    z64 = jnp.zeros((NP, HID), f32)

    phase_a = pl.kernel(
        _sc_phase_a_body,
        out_type=[
            jax.ShapeDtypeStruct((EP, 16), f32),
            jax.ShapeDtypeStruct((EP, 16), f32),
            jax.ShapeDtypeStruct((2, NP, 16), f32),
            jax.ShapeDtypeStruct((2, NP, 16), f32),
        ],
        mesh=mesh,
        compiler_params=pltpu.CompilerParams(use_tc_tiling_on_sc=False),
        scratch_types=[
            pltpu.VMEM((4, 2, CHUNK), jnp.int32),
            pltpu.VMEM((4, CHUNK, 16), f32),
            pltpu.VMEM((4, CHUNK, 16), f32),
            pltpu.VMEM((4, CHUNK, 16), f32),
            pltpu.VMEM_SHARED((NP, 16), f32),
        ] + [pltpu.SemaphoreType.DMA] * 16,
    )
    ee_qa, ee_aq, dp_qa, dp_aq = phase_a(
        epa_qa, epa_aq, al_s_qa, al_d_qa, al_s_aq, al_d_aq, z16)

    den_qa = dp_qa[0] + dp_qa[1]
    den_aq = dp_aq[0] + dp_aq[1]

    phase_b = pl.kernel(
        _sc_phase_b_body,
        out_type=[
            jax.ShapeDtypeStruct((2, NP, HID), f32),
            jax.ShapeDtypeStruct((2, NP, HID), f32),
        ],
        mesh=mesh,
        compiler_params=pltpu.CompilerParams(use_tc_tiling_on_sc=False,
                                             needs_layout_passes=False),
        scratch_types=[
            pltpu.VMEM((4, 2, CHUNK_B), jnp.int32),
            pltpu.VMEM((2, CHUNK_B, 16), f32),
            pltpu.VMEM((2, CHUNK_B, 16), f32),
            pltpu.VMEM((CHUNK_B, 16), f32),
            pltpu.VMEM((2, CHUNK_B, HEADS * HID), jnp.bfloat16),
            pltpu.VMEM((2, CHUNK_B, HID), f32),
            pltpu.VMEM_SHARED((NP, HID), f32),
        ] + [pltpu.SemaphoreType.DMA] * 8,
    )
    op_qa, op_aq = phase_b(
        ep_qa, ep_aq, ee_qa, ee_aq, den_qa, den_aq,
        hs_qa, hs_aq, z64)

    tc2 = pl.pallas_call(
        _tc2_body,
        grid=(GRID_N,),
        in_specs=[
            pl.BlockSpec((2, BN, HID), lambda i: (0, i, 0)),
            pl.BlockSpec((2, BN, HID), lambda i: (0, i, 0)),
            _full((1, HID)), _full((1, HID)),
            _full((2 * HID, NC_OUT)), _full((1, NC_OUT)),
        ],
        out_specs=pl.BlockSpec((BN, NC_OUT), lambda i: (i, 0)),
        out_shape=jax.ShapeDtypeStruct((NP, NC_OUT), f32),
    )
    preds = tc2(op_qa, op_aq, bias_qa_eff.reshape(1, HID),
                bias_aq_eff.reshape(1, HID), w_eff, b_out.reshape(1, NC_OUT))
    return (preds[:N], ew_qa, ew_aq)


# speedup vs baseline: 1.1757x; 1.0985x over previous
import functools
import jax
import jax.numpy as jnp
from jax import lax
from jax.experimental import pallas as pl
from jax.experimental.pallas import tpu as pltpu, tpu_sc as plsc

N = 10000
E = 160000
D_IN = 128
HID = 64
HEADS = 8
NC_OUT = 4

NP = 10240
EP = 163840
PAD_NODE = N

NUM_TILES = 32
EDGES_PER_TILE = EP // NUM_TILES
CHUNK = 128
NCHUNKS = EDGES_PER_TILE // CHUNK
CHUNK_B = 80
NCHUNKS_B = EDGES_PER_TILE // CHUNK_B
NCB0 = 80
NCB1 = 48
ROWS_PER_TILE = NP // 16

BN = 256
GRID_N = NP // BN


def _tc1_body(xq, xa, wsqa, asqa, wdqa, adqa, wsaq, asaq, wdaq, adaq,
              hs_qa, al_s_qa, al_d_qa, hs_aq, al_s_aq, al_d_aq):
    xqb = xq[...]
    xab = xa[...]
    hq = jnp.dot(xqb, wsqa[...], preferred_element_type=jnp.float32)
    hs_qa[...] = hq.astype(jnp.bfloat16)
    al_s_qa[...] = jnp.dot(hq, asqa[...], preferred_element_type=jnp.float32)
    vd_qa = jnp.dot(wdqa[...], adqa[...], preferred_element_type=jnp.float32)
    al_d_qa[...] = jnp.dot(xab, vd_qa, preferred_element_type=jnp.float32)
    ha = jnp.dot(xab, wsaq[...], preferred_element_type=jnp.float32)
    hs_aq[...] = ha.astype(jnp.bfloat16)
    al_s_aq[...] = jnp.dot(ha, asaq[...], preferred_element_type=jnp.float32)
    vd_aq = jnp.dot(wdaq[...], adaq[...], preferred_element_type=jnp.float32)
    al_d_aq[...] = jnp.dot(xqb, vd_aq, preferred_element_type=jnp.float32)


def _sc_phase_a_body(epa_qa, epa_aq, asq, adq, asa, ada, z16,
                     ee_qa, ee_aq, dp_qa, dp_aq,
                     idx4, aa4, bb4, ee4, den_sh,
                     sa0, sa1, sa2, sa3, sb0, sb1, sb2, sb3,
                     sw0, sw1, sw2, sw3, ss0, ss1, ss2, ss3):
    cid = lax.axis_index("c")
    sid = lax.axis_index("s")
    wid = cid * 16 + sid
    r0 = sid * ROWS_PER_TILE
    sem_a = (sa0, sa1, sa2, sa3)
    sem_b = (sb0, sb1, sb2, sb3)
    sem_w = (sw0, sw1, sw2, sw3)
    sem_s = (ss0, ss1, ss2, ss3)

    for epa, asrc, adst, ee_o, dp_o in (
        (epa_qa, asq, adq, ee_qa, dp_qa),
        (epa_aq, asa, ada, ee_aq, dp_aq),
    ):
        pltpu.sync_copy(z16.at[pl.ds(r0, ROWS_PER_TILE)],
                        den_sh.at[pl.ds(r0, ROWS_PER_TILE)])
        plsc.subcore_barrier()

        def fire(k, b, epa=epa, asrc=asrc, adst=adst):
            g = wid * NCHUNKS + k
            pltpu.sync_copy(epa.at[g], idx4.at[b])
            pltpu.make_async_copy(asrc.at[idx4.at[b].at[0]], aa4.at[b],
                                  sem_a[b]).start()
            pltpu.make_async_copy(adst.at[idx4.at[b].at[1]], bb4.at[b],
                                  sem_b[b]).start()

        def drain(k, b, ee_o=ee_o):
            base = (wid * NCHUNKS + k) * CHUNK
            pltpu.make_async_copy(ee4.at[b], ee_o.at[pl.ds(base, CHUNK)],
                                  sem_w[b]).wait()
            pltpu.make_async_copy(ee4.at[b], den_sh.at[idx4.at[b].at[1]],
                                  sem_s[b]).wait()

        def process(k, b, asrc=asrc, adst=adst, ee_o=ee_o):
            base = (wid * NCHUNKS + k) * CHUNK
            pltpu.make_async_copy(asrc.at[idx4.at[b].at[0]], aa4.at[b],
                                  sem_a[b]).wait()
            pltpu.make_async_copy(adst.at[idx4.at[b].at[1]], bb4.at[b],
                                  sem_b[b]).wait()

            @pl.when(k > 0)
            def _():
                drain(k - 1, (b + 3) % 4)

            @pl.when(k + 3 < NCHUNKS)
            def _():
                fire(k + 3, (b + 3) % 4)

            def row(i, _):
                sv = aa4[b, i, :] + bb4[b, i, :]
                e = jnp.maximum(sv, 0.2 * sv)
                ee4[b, i, :] = jnp.exp(e)
                return 0
            lax.fori_loop(0, CHUNK, row, 0)
            pltpu.make_async_copy(ee4.at[b], ee_o.at[pl.ds(base, CHUNK)],
                                  sem_w[b]).start()
            pltpu.make_async_copy(ee4.at[b], den_sh.at[idx4.at[b].at[1]],
                                  sem_s[b]).start(add=True)

        fire(0, 0)
        fire(1, 1)
        fire(2, 2)

        def quad(kk, _):
            for b in range(4):
                process(4 * kk + b, b)
            return 0
        lax.fori_loop(0, NCHUNKS // 4, quad, 0)
        drain(NCHUNKS - 1, 3)

        plsc.subcore_barrier()
        pltpu.sync_copy(den_sh.at[pl.ds(r0, ROWS_PER_TILE)],
                        dp_o.at[cid, pl.ds(r0, ROWS_PER_TILE)])
        plsc.subcore_barrier()


def _sc_phase_b_body(ep_qa, ep_aq, ee_qa, ee_aq, den_qa, den_aq,
                     hs_qa, hs_aq, z64,
                     op_qa, op_aq,
                     idx4, ee2, den2, wbuf, hs2, msg2, out_sh,
                     se0, se1, sd0, sd1, sh0, sh1, ss0, ss1):
    cid = lax.axis_index("c")
    sid = lax.axis_index("s")
    r0 = sid * ROWS_PER_TILE
    nc = jnp.where(cid == 0, NCB0, NCB1)
    gbase = jnp.where(cid == 0, sid * NCB0, 16 * NCB0 + sid * NCB1)
    sem_e = (se0, se1)
    sem_d = (sd0, sd1)
    sem_h = (sh0, sh1)
    sem_s = (ss0, ss1)

    for ep, ee, den, hs, op_o in (
        (ep_qa, ee_qa, den_qa, hs_qa, op_qa),
        (ep_aq, ee_aq, den_aq, hs_aq, op_aq),
    ):
        pltpu.sync_copy(z64.at[pl.ds(r0, ROWS_PER_TILE)],
                        out_sh.at[pl.ds(r0, ROWS_PER_TILE)])
        plsc.subcore_barrier()

        def fire_idx(k, b, ep=ep):
            pltpu.sync_copy(ep.at[gbase + k], idx4.at[b])

        def fire_hs(k, b, bh, hs=hs):
            pltpu.make_async_copy(hs.at[idx4.at[b].at[0]], hs2.at[bh],
                                  sem_h[bh]).start()

        def fire_near(k, b, b2, ee=ee, den=den):
            base = (gbase + k) * CHUNK_B
            pltpu.make_async_copy(den.at[idx4.at[b].at[1]], den2.at[b2],
                                  sem_d[b2]).start()
            pltpu.make_async_copy(ee.at[pl.ds(base, CHUNK_B)], ee2.at[b2],
                                  sem_e[b2]).start()

        def wait_scatter(b, b2):
            pltpu.make_async_copy(msg2.at[b2], out_sh.at[idx4.at[b].at[1]],
                                  sem_s[b2]).wait()

        def process(k, b, b2, ee=ee, den=den, hs=hs):
            pltpu.make_async_copy(den.at[idx4.at[b].at[1]], den2.at[b2],
                                  sem_d[b2]).wait()
            pltpu.make_async_copy(ee.at[pl.ds(0, CHUNK_B)], ee2.at[b2],
                                  sem_e[b2]).wait()

            def wrow(i, _):
                wbuf[i, :] = ee2[b2, i, :] * 0.125 / (den2[b2, i, :] + 1e-16)
                return 0
            lax.fori_loop(0, CHUNK_B, wrow, 0)

            @pl.when(k > 0)
            def _():
                wait_scatter((b + 3) % 4, (b2 + 1) % 2)

            @pl.when(k + 3 < nc)
            def _():
                fire_idx(k + 3, (b + 3) % 4)

            @pl.when(k + 1 < nc)
            def _():
                fire_hs(k + 1, (b + 1) % 4, (b2 + 1) % 2)
                fire_near(k + 1, (b + 1) % 4, (b2 + 1) % 2)

            pltpu.make_async_copy(hs.at[idx4.at[b].at[0]], hs2.at[b2],
                                  sem_h[b2]).wait()

            def edge(e_, _):
                wrow_ = wbuf[e_, :]
                ws = [wrow_[h] for h in range(8)]
                acc = [None] * 4
                for h in range(8):
                    for j2 in range(2):
                        raw = hs2[b2, e_, pl.ds(h * 64 + j2 * 32, 32)]
                        ti = plsc.bitcast(raw, jnp.int32)
                        fe = plsc.bitcast(ti << 16, jnp.float32)
                        fo = plsc.bitcast(ti & jnp.int32(-65536), jnp.float32)
                        te = ws[h] * fe
                        to = ws[h] * fo
                        qe, qo = 2 * j2, 2 * j2 + 1
                        acc[qe] = te if h == 0 else acc[qe] + te
                        acc[qo] = to if h == 0 else acc[qo] + to
                for q in range(4):
                    msg2[b2, e_, pl.ds(q * 16, 16)] = acc[q]
                return 0
            lax.fori_loop(0, CHUNK_B, edge, 0)
            pltpu.make_async_copy(msg2.at[b2], out_sh.at[idx4.at[b].at[1]],
                                  sem_s[b2]).start(add=True)

        fire_idx(0, 0)
        fire_idx(1, 1)
        fire_idx(2, 2)
        fire_hs(0, 0, 0)
        fire_near(0, 0, 0)

        def quad(kk, _):
            for b in range(4):
                process(4 * kk + b, b, b % 2)
            return 0
        lax.fori_loop(0, nc // 4, quad, 0)
        wait_scatter(3, 1)

        plsc.subcore_barrier()
        pltpu.sync_copy(out_sh.at[pl.ds(r0, ROWS_PER_TILE)],
                        op_o.at[cid, pl.ds(r0, ROWS_PER_TILE)])
        plsc.subcore_barrier()


def _tc2_body(opqa, opaq, bqa, baq, wo, bo, out):
    f1 = opaq[0] + opaq[1] + baq[...]
    f2 = opqa[0] + opqa[1] + bqa[...]
    out[...] = (jnp.dot(f1, wo[0:64, :], preferred_element_type=jnp.float32)
                + jnp.dot(f2, wo[64:128, :], preferred_element_type=jnp.float32)
                + bo[...])


def _full(shape):
    return pl.BlockSpec(shape, lambda i: (0,) * len(shape))


@jax.jit
def kernel(x_question, x_answer, edge_index_qa, edge_index_aq,
           W_src_qa, W_dst_qa, att_src_qa, att_dst_qa, bias_qa,
           W_src_aq, W_dst_aq, att_src_aq, att_dst_aq, bias_aq,
           W_out, b_out, ew_qa, ew_aq):
    f32 = jnp.float32
    eye = jnp.eye(HEADS, dtype=f32)

    def amat(att):
        a = (att[:, :, None] * eye[:, None, :]).reshape(HEADS * HID, HEADS)
        return jnp.pad(a, ((0, 0), (0, 16 - HEADS)))

    a_s_qa, a_d_qa = amat(att_src_qa), amat(att_dst_qa)
    a_s_aq, a_d_aq = amat(att_src_aq), amat(att_dst_aq)

    l16 = jnp.arange(16)
    perm = jnp.concatenate([2 * l16, 2 * l16 + 1, 32 + 2 * l16, 33 + 2 * l16])
    w_eff = jnp.concatenate([W_out[:64][perm], W_out[64:][perm]], axis=0)
    bias_qa_eff = bias_qa[perm]
    bias_aq_eff = bias_aq[perm]

    xq = jnp.pad(x_question, ((0, NP - N), (0, 0)))
    xa = jnp.pad(x_answer, ((0, NP - N), (0, 0)))

    pad_idx = jnp.full((EP - E,), PAD_NODE, dtype=jnp.int32)
    es_qa = jnp.concatenate([edge_index_qa[0].astype(jnp.int32), pad_idx])
    ed_qa = jnp.concatenate([edge_index_qa[1].astype(jnp.int32), pad_idx])
    es_aq = jnp.concatenate([edge_index_aq[0].astype(jnp.int32), pad_idx])
    ed_aq = jnp.concatenate([edge_index_aq[1].astype(jnp.int32), pad_idx])
    ep_qa = jnp.stack([es_qa.reshape(-1, CHUNK_B), ed_qa.reshape(-1, CHUNK_B)], 1)
    ep_aq = jnp.stack([es_aq.reshape(-1, CHUNK_B), ed_aq.reshape(-1, CHUNK_B)], 1)
    epa_qa = jnp.stack([es_qa.reshape(-1, CHUNK), ed_qa.reshape(-1, CHUNK)], 1)
    epa_aq = jnp.stack([es_aq.reshape(-1, CHUNK), ed_aq.reshape(-1, CHUNK)], 1)

    tc1 = pl.pallas_call(
        _tc1_body,
        grid=(GRID_N,),
        in_specs=[
            pl.BlockSpec((BN, D_IN), lambda i: (i, 0)),
            pl.BlockSpec((BN, D_IN), lambda i: (i, 0)),
            _full((D_IN, HEADS * HID)), _full((HEADS * HID, 16)),
            _full((D_IN, HEADS * HID)), _full((HEADS * HID, 16)),
            _full((D_IN, HEADS * HID)), _full((HEADS * HID, 16)),
            _full((D_IN, HEADS * HID)), _full((HEADS * HID, 16)),
        ],
        out_specs=[
            pl.BlockSpec((BN, HEADS * HID), lambda i: (i, 0)),
            pl.BlockSpec((BN, 16), lambda i: (i, 0)),
            pl.BlockSpec((BN, 16), lambda i: (i, 0)),
            pl.BlockSpec((BN, HEADS * HID), lambda i: (i, 0)),
            pl.BlockSpec((BN, 16), lambda i: (i, 0)),
            pl.BlockSpec((BN, 16), lambda i: (i, 0)),
        ],
        out_shape=[
            jax.ShapeDtypeStruct((NP, HEADS * HID), jnp.bfloat16),
            jax.ShapeDtypeStruct((NP, 16), f32),
            jax.ShapeDtypeStruct((NP, 16), f32),
            jax.ShapeDtypeStruct((NP, HEADS * HID), jnp.bfloat16),
            jax.ShapeDtypeStruct((NP, 16), f32),
            jax.ShapeDtypeStruct((NP, 16), f32),
        ],
    )
    hs_qa, al_s_qa, al_d_qa, hs_aq, al_s_aq, al_d_aq = tc1(
        xq, xa, W_src_qa, a_s_qa, W_dst_qa, a_d_qa,
        W_src_aq, a_s_aq, W_dst_aq, a_d_aq)

    mesh = plsc.VectorSubcoreMesh(core_axis_name="c", subcore_axis_name="s")
    z16 = jnp.zeros((NP, 16), f32)
    z64 = jnp.zeros((NP, HID), f32)

    phase_a = pl.kernel(
        _sc_phase_a_body,
        out_type=[
            jax.ShapeDtypeStruct((EP, 16), f32),
            jax.ShapeDtypeStruct((EP, 16), f32),
            jax.ShapeDtypeStruct((2, NP, 16), f32),
            jax.ShapeDtypeStruct((2, NP, 16), f32),
        ],
        mesh=mesh,
        compiler_params=pltpu.CompilerParams(use_tc_tiling_on_sc=False),
        scratch_types=[
            pltpu.VMEM((4, 2, CHUNK), jnp.int32),
            pltpu.VMEM((4, CHUNK, 16), f32),
            pltpu.VMEM((4, CHUNK, 16), f32),
            pltpu.VMEM((4, CHUNK, 16), f32),
            pltpu.VMEM_SHARED((NP, 16), f32),
        ] + [pltpu.SemaphoreType.DMA] * 16,
    )
    ee_qa, ee_aq, dp_qa, dp_aq = phase_a(
        epa_qa, epa_aq, al_s_qa, al_d_qa, al_s_aq, al_d_aq, z16)

    den_qa = dp_qa[0] + dp_qa[1]
    den_aq = dp_aq[0] + dp_aq[1]

    phase_b = pl.kernel(
        _sc_phase_b_body,
        out_type=[
            jax.ShapeDtypeStruct((2, NP, HID), f32),
            jax.ShapeDtypeStruct((2, NP, HID), f32),
        ],
        mesh=mesh,
        compiler_params=pltpu.CompilerParams(use_tc_tiling_on_sc=False,
                                             needs_layout_passes=False),
        scratch_types=[
            pltpu.VMEM((4, 2, CHUNK_B), jnp.int32),
            pltpu.VMEM((2, CHUNK_B, 16), f32),
            pltpu.VMEM((2, CHUNK_B, 16), f32),
            pltpu.VMEM((CHUNK_B, 16), f32),
            pltpu.VMEM((2, CHUNK_B, HEADS * HID), jnp.bfloat16),
            pltpu.VMEM((2, CHUNK_B, HID), f32),
            pltpu.VMEM_SHARED((NP, HID), f32),
        ] + [pltpu.SemaphoreType.DMA] * 8,
    )
    op_qa, op_aq = phase_b(
        ep_qa, ep_aq, ee_qa, ee_aq, den_qa, den_aq,
        hs_qa, hs_aq, z64)

    tc2 = pl.pallas_call(
        _tc2_body,
        grid=(GRID_N,),
        in_specs=[
            pl.BlockSpec((2, BN, HID), lambda i: (0, i, 0)),
            pl.BlockSpec((2, BN, HID), lambda i: (0, i, 0)),
            _full((1, HID)), _full((1, HID)),
            _full((2 * HID, NC_OUT)), _full((1, NC_OUT)),
        ],
        out_specs=pl.BlockSpec((BN, NC_OUT), lambda i: (i, 0)),
        out_shape=jax.ShapeDtypeStruct((NP, NC_OUT), f32),
    )
    preds = tc2(op_qa, op_aq, bias_qa_eff.reshape(1, HID),
                bias_aq_eff.reshape(1, HID), w_eff, b_out.reshape(1, NC_OUT))
    return (preds[:N], ew_qa, ew_aq)


# speedup vs baseline: 1.1806x; 1.0041x over previous
import functools
import jax
import jax.numpy as jnp
from jax import lax
from jax.experimental import pallas as pl
from jax.experimental.pallas import tpu as pltpu, tpu_sc as plsc

N = 10000
E = 160000
D_IN = 128
HID = 64
HEADS = 8
NC_OUT = 4

NP = 10240
EP = 163840
PAD_NODE = N

NUM_TILES = 32
EDGES_PER_TILE = EP // NUM_TILES
CHUNK = 128
NCHUNKS = EDGES_PER_TILE // CHUNK
CHUNK_B = 80
NCHUNKS_B = EDGES_PER_TILE // CHUNK_B
NCB0 = 84
NCB1 = 44
ROWS_PER_TILE = NP // 16

BN = 256
GRID_N = NP // BN


def _tc1_body(xq, xa, wsqa, asqa, wdqa, adqa, wsaq, asaq, wdaq, adaq,
              hs_qa, al_s_qa, al_d_qa, hs_aq, al_s_aq, al_d_aq):
    xqb = xq[...]
    xab = xa[...]
    hq = jnp.dot(xqb, wsqa[...], preferred_element_type=jnp.float32)
    hs_qa[...] = hq.astype(jnp.bfloat16)
    al_s_qa[...] = jnp.dot(hq, asqa[...], preferred_element_type=jnp.float32)
    vd_qa = jnp.dot(wdqa[...], adqa[...], preferred_element_type=jnp.float32)
    al_d_qa[...] = jnp.dot(xab, vd_qa, preferred_element_type=jnp.float32)
    ha = jnp.dot(xab, wsaq[...], preferred_element_type=jnp.float32)
    hs_aq[...] = ha.astype(jnp.bfloat16)
    al_s_aq[...] = jnp.dot(ha, asaq[...], preferred_element_type=jnp.float32)
    vd_aq = jnp.dot(wdaq[...], adaq[...], preferred_element_type=jnp.float32)
    al_d_aq[...] = jnp.dot(xqb, vd_aq, preferred_element_type=jnp.float32)


def _sc_phase_a_body(epa_qa, epa_aq, asq, adq, asa, ada, z16,
                     ee_qa, ee_aq, dp_qa, dp_aq,
                     idx4, aa4, bb4, ee4, den_sh,
                     sa0, sa1, sa2, sa3, sb0, sb1, sb2, sb3,
                     sw0, sw1, sw2, sw3, ss0, ss1, ss2, ss3):
    cid = lax.axis_index("c")
    sid = lax.axis_index("s")
    wid = cid * 16 + sid
    r0 = sid * ROWS_PER_TILE
    sem_a = (sa0, sa1, sa2, sa3)
    sem_b = (sb0, sb1, sb2, sb3)
    sem_w = (sw0, sw1, sw2, sw3)
    sem_s = (ss0, ss1, ss2, ss3)

    for epa, asrc, adst, ee_o, dp_o in (
        (epa_qa, asq, adq, ee_qa, dp_qa),
        (epa_aq, asa, ada, ee_aq, dp_aq),
    ):
        pltpu.sync_copy(z16.at[pl.ds(r0, ROWS_PER_TILE)],
                        den_sh.at[pl.ds(r0, ROWS_PER_TILE)])
        plsc.subcore_barrier()

        def fire(k, b, epa=epa, asrc=asrc, adst=adst):
            g = wid * NCHUNKS + k
            pltpu.sync_copy(epa.at[g], idx4.at[b])
            pltpu.make_async_copy(asrc.at[idx4.at[b].at[0]], aa4.at[b],
                                  sem_a[b]).start()
            pltpu.make_async_copy(adst.at[idx4.at[b].at[1]], bb4.at[b],
                                  sem_b[b]).start()

        def drain(k, b, ee_o=ee_o):
            base = (wid * NCHUNKS + k) * CHUNK
            pltpu.make_async_copy(ee4.at[b], ee_o.at[pl.ds(base, CHUNK)],
                                  sem_w[b]).wait()
            pltpu.make_async_copy(ee4.at[b], den_sh.at[idx4.at[b].at[1]],
                                  sem_s[b]).wait()

        def process(k, b, asrc=asrc, adst=adst, ee_o=ee_o):
            base = (wid * NCHUNKS + k) * CHUNK
            pltpu.make_async_copy(asrc.at[idx4.at[b].at[0]], aa4.at[b],
                                  sem_a[b]).wait()
            pltpu.make_async_copy(adst.at[idx4.at[b].at[1]], bb4.at[b],
                                  sem_b[b]).wait()

            @pl.when(k > 0)
            def _():
                drain(k - 1, (b + 3) % 4)

            @pl.when(k + 3 < NCHUNKS)
            def _():
                fire(k + 3, (b + 3) % 4)

            def row(i, _):
                sv = aa4[b, i, :] + bb4[b, i, :]
                e = jnp.maximum(sv, 0.2 * sv)
                ee4[b, i, :] = jnp.exp(e)
                return 0
            lax.fori_loop(0, CHUNK, row, 0)
            pltpu.make_async_copy(ee4.at[b], ee_o.at[pl.ds(base, CHUNK)],
                                  sem_w[b]).start()
            pltpu.make_async_copy(ee4.at[b], den_sh.at[idx4.at[b].at[1]],
                                  sem_s[b]).start(add=True)

        fire(0, 0)
        fire(1, 1)
        fire(2, 2)

        def quad(kk, _):
            for b in range(4):
                process(4 * kk + b, b)
            return 0
        lax.fori_loop(0, NCHUNKS // 4, quad, 0)
        drain(NCHUNKS - 1, 3)

        plsc.subcore_barrier()
        pltpu.sync_copy(den_sh.at[pl.ds(r0, ROWS_PER_TILE)],
                        dp_o.at[cid, pl.ds(r0, ROWS_PER_TILE)])
        plsc.subcore_barrier()


def _sc_phase_b_body(ep_qa, ep_aq, ee_qa, ee_aq, den_qa, den_aq,
                     hs_qa, hs_aq, z64,
                     op_qa, op_aq,
                     idx4, ee2, den2, wbuf, hs2, msg2, out_sh,
                     se0, se1, sd0, sd1, sh0, sh1, ss0, ss1):
    cid = lax.axis_index("c")
    sid = lax.axis_index("s")
    r0 = sid * ROWS_PER_TILE
    nc = jnp.where(cid == 0, NCB0, NCB1)
    gbase = jnp.where(cid == 0, sid * NCB0, 16 * NCB0 + sid * NCB1)
    sem_e = (se0, se1)
    sem_d = (sd0, sd1)
    sem_h = (sh0, sh1)
    sem_s = (ss0, ss1)

    for ep, ee, den, hs, op_o in (
        (ep_qa, ee_qa, den_qa, hs_qa, op_qa),
        (ep_aq, ee_aq, den_aq, hs_aq, op_aq),
    ):
        pltpu.sync_copy(z64.at[pl.ds(r0, ROWS_PER_TILE)],
                        out_sh.at[pl.ds(r0, ROWS_PER_TILE)])
        plsc.subcore_barrier()

        def fire_idx(k, b, ep=ep):
            pltpu.sync_copy(ep.at[gbase + k], idx4.at[b])

        def fire_hs(k, b, bh, hs=hs):
            pltpu.make_async_copy(hs.at[idx4.at[b].at[0]], hs2.at[bh],
                                  sem_h[bh]).start()

        def fire_near(k, b, b2, ee=ee, den=den):
            base = (gbase + k) * CHUNK_B
            pltpu.make_async_copy(den.at[idx4.at[b].at[1]], den2.at[b2],
                                  sem_d[b2]).start()
            pltpu.make_async_copy(ee.at[pl.ds(base, CHUNK_B)], ee2.at[b2],
                                  sem_e[b2]).start()

        def wait_scatter(b, b2):
            pltpu.make_async_copy(msg2.at[b2], out_sh.at[idx4.at[b].at[1]],
                                  sem_s[b2]).wait()

        def process(k, b, b2, ee=ee, den=den, hs=hs):
            pltpu.make_async_copy(den.at[idx4.at[b].at[1]], den2.at[b2],
                                  sem_d[b2]).wait()
            pltpu.make_async_copy(ee.at[pl.ds(0, CHUNK_B)], ee2.at[b2],
                                  sem_e[b2]).wait()

            def wrow(i, _):
                wbuf[i, :] = ee2[b2, i, :] * 0.125 / (den2[b2, i, :] + 1e-16)
                return 0
            lax.fori_loop(0, CHUNK_B, wrow, 0)

            @pl.when(k > 0)
            def _():
                wait_scatter((b + 3) % 4, (b2 + 1) % 2)

            @pl.when(k + 3 < nc)
            def _():
                fire_idx(k + 3, (b + 3) % 4)

            @pl.when(k + 1 < nc)
            def _():
                fire_hs(k + 1, (b + 1) % 4, (b2 + 1) % 2)
                fire_near(k + 1, (b + 1) % 4, (b2 + 1) % 2)

            pltpu.make_async_copy(hs.at[idx4.at[b].at[0]], hs2.at[b2],
                                  sem_h[b2]).wait()

            def edge(e_, _):
                wrow_ = wbuf[e_, :]
                ws = [wrow_[h] for h in range(8)]
                acc = [None] * 4
                for h in range(8):
                    for j2 in range(2):
                        raw = hs2[b2, e_, pl.ds(h * 64 + j2 * 32, 32)]
                        ti = plsc.bitcast(raw, jnp.int32)
                        fe = plsc.bitcast(ti << 16, jnp.float32)
                        fo = plsc.bitcast(ti & jnp.int32(-65536), jnp.float32)
                        te = ws[h] * fe
                        to = ws[h] * fo
                        qe, qo = 2 * j2, 2 * j2 + 1
                        acc[qe] = te if h == 0 else acc[qe] + te
                        acc[qo] = to if h == 0 else acc[qo] + to
                for q in range(4):
                    msg2[b2, e_, pl.ds(q * 16, 16)] = acc[q]
                return 0
            lax.fori_loop(0, CHUNK_B, edge, 0)
            pltpu.make_async_copy(msg2.at[b2], out_sh.at[idx4.at[b].at[1]],
                                  sem_s[b2]).start(add=True)

        fire_idx(0, 0)
        fire_idx(1, 1)
        fire_idx(2, 2)
        fire_hs(0, 0, 0)
        fire_near(0, 0, 0)

        def quad(kk, _):
            for b in range(4):
                process(4 * kk + b, b, b % 2)
            return 0
        lax.fori_loop(0, nc // 4, quad, 0)
        wait_scatter(3, 1)

        plsc.subcore_barrier()
        pltpu.sync_copy(out_sh.at[pl.ds(r0, ROWS_PER_TILE)],
                        op_o.at[cid, pl.ds(r0, ROWS_PER_TILE)])
        plsc.subcore_barrier()


def _tc2_body(opqa, opaq, bqa, baq, wo, bo, out):
    f1 = opaq[0] + opaq[1] + baq[...]
    f2 = opqa[0] + opqa[1] + bqa[...]
    out[...] = (jnp.dot(f1, wo[0:64, :], preferred_element_type=jnp.float32)
                + jnp.dot(f2, wo[64:128, :], preferred_element_type=jnp.float32)
                + bo[...])


def _full(shape):
    return pl.BlockSpec(shape, lambda i: (0,) * len(shape))


@jax.jit
def kernel(x_question, x_answer, edge_index_qa, edge_index_aq,
           W_src_qa, W_dst_qa, att_src_qa, att_dst_qa, bias_qa,
           W_src_aq, W_dst_aq, att_src_aq, att_dst_aq, bias_aq,
           W_out, b_out, ew_qa, ew_aq):
    f32 = jnp.float32
    eye = jnp.eye(HEADS, dtype=f32)

    def amat(att):
        a = (att[:, :, None] * eye[:, None, :]).reshape(HEADS * HID, HEADS)
        return jnp.pad(a, ((0, 0), (0, 16 - HEADS)))

    a_s_qa, a_d_qa = amat(att_src_qa), amat(att_dst_qa)
    a_s_aq, a_d_aq = amat(att_src_aq), amat(att_dst_aq)

    l16 = jnp.arange(16)
    perm = jnp.concatenate([2 * l16, 2 * l16 + 1, 32 + 2 * l16, 33 + 2 * l16])
    w_eff = jnp.concatenate([W_out[:64][perm], W_out[64:][perm]], axis=0)
    bias_qa_eff = bias_qa[perm]
    bias_aq_eff = bias_aq[perm]

    xq = jnp.pad(x_question, ((0, NP - N), (0, 0)))
    xa = jnp.pad(x_answer, ((0, NP - N), (0, 0)))

    pad_idx = jnp.full((EP - E,), PAD_NODE, dtype=jnp.int32)
    es_qa = jnp.concatenate([edge_index_qa[0].astype(jnp.int32), pad_idx])
    ed_qa = jnp.concatenate([edge_index_qa[1].astype(jnp.int32), pad_idx])
    es_aq = jnp.concatenate([edge_index_aq[0].astype(jnp.int32), pad_idx])
    ed_aq = jnp.concatenate([edge_index_aq[1].astype(jnp.int32), pad_idx])
    ep_qa = jnp.stack([es_qa.reshape(-1, CHUNK_B), ed_qa.reshape(-1, CHUNK_B)], 1)
    ep_aq = jnp.stack([es_aq.reshape(-1, CHUNK_B), ed_aq.reshape(-1, CHUNK_B)], 1)
    epa_qa = jnp.stack([es_qa.reshape(-1, CHUNK), ed_qa.reshape(-1, CHUNK)], 1)
    epa_aq = jnp.stack([es_aq.reshape(-1, CHUNK), ed_aq.reshape(-1, CHUNK)], 1)

    tc1 = pl.pallas_call(
        _tc1_body,
        grid=(GRID_N,),
        in_specs=[
            pl.BlockSpec((BN, D_IN), lambda i: (i, 0)),
            pl.BlockSpec((BN, D_IN), lambda i: (i, 0)),
            _full((D_IN, HEADS * HID)), _full((HEADS * HID, 16)),
            _full((D_IN, HEADS * HID)), _full((HEADS * HID, 16)),
            _full((D_IN, HEADS * HID)), _full((HEADS * HID, 16)),
            _full((D_IN, HEADS * HID)), _full((HEADS * HID, 16)),
        ],
        out_specs=[
            pl.BlockSpec((BN, HEADS * HID), lambda i: (i, 0)),
            pl.BlockSpec((BN, 16), lambda i: (i, 0)),
            pl.BlockSpec((BN, 16), lambda i: (i, 0)),
            pl.BlockSpec((BN, HEADS * HID), lambda i: (i, 0)),
            pl.BlockSpec((BN, 16), lambda i: (i, 0)),
            pl.BlockSpec((BN, 16), lambda i: (i, 0)),
        ],
        out_shape=[
            jax.ShapeDtypeStruct((NP, HEADS * HID), jnp.bfloat16),
            jax.ShapeDtypeStruct((NP, 16), f32),
            jax.ShapeDtypeStruct((NP, 16), f32),
            jax.ShapeDtypeStruct((NP, HEADS * HID), jnp.bfloat16),
            jax.ShapeDtypeStruct((NP, 16), f32),
            jax.ShapeDtypeStruct((NP, 16), f32),
        ],
    )
    hs_qa, al_s_qa, al_d_qa, hs_aq, al_s_aq, al_d_aq = tc1(
        xq, xa, W_src_qa, a_s_qa, W_dst_qa, a_d_qa,
        W_src_aq, a_s_aq, W_dst_aq, a_d_aq)

    mesh = plsc.VectorSubcoreMesh(core_axis_name="c", subcore_axis_name="s")
    z16 = jnp.zeros((NP, 16), f32)
    z64 = jnp.zeros((NP, HID), f32)

    phase_a = pl.kernel(
        _sc_phase_a_body,
        out_type=[
            jax.ShapeDtypeStruct((EP, 16), f32),
            jax.ShapeDtypeStruct((EP, 16), f32),
            jax.ShapeDtypeStruct((2, NP, 16), f32),
            jax.ShapeDtypeStruct((2, NP, 16), f32),
        ],
        mesh=mesh,
        compiler_params=pltpu.CompilerParams(use_tc_tiling_on_sc=False),
        scratch_types=[
            pltpu.VMEM((4, 2, CHUNK), jnp.int32),
            pltpu.VMEM((4, CHUNK, 16), f32),
            pltpu.VMEM((4, CHUNK, 16), f32),
            pltpu.VMEM((4, CHUNK, 16), f32),
            pltpu.VMEM_SHARED((NP, 16), f32),
        ] + [pltpu.SemaphoreType.DMA] * 16,
    )
    ee_qa, ee_aq, dp_qa, dp_aq = phase_a(
        epa_qa, epa_aq, al_s_qa, al_d_qa, al_s_aq, al_d_aq, z16)

    den_qa = dp_qa[0] + dp_qa[1]
    den_aq = dp_aq[0] + dp_aq[1]

    phase_b = pl.kernel(
        _sc_phase_b_body,
        out_type=[
            jax.ShapeDtypeStruct((2, NP, HID), f32),
            jax.ShapeDtypeStruct((2, NP, HID), f32),
        ],
        mesh=mesh,
        compiler_params=pltpu.CompilerParams(use_tc_tiling_on_sc=False,
                                             needs_layout_passes=False),
        scratch_types=[
            pltpu.VMEM((4, 2, CHUNK_B), jnp.int32),
            pltpu.VMEM((2, CHUNK_B, 16), f32),
            pltpu.VMEM((2, CHUNK_B, 16), f32),
            pltpu.VMEM((CHUNK_B, 16), f32),
            pltpu.VMEM((2, CHUNK_B, HEADS * HID), jnp.bfloat16),
            pltpu.VMEM((2, CHUNK_B, HID), f32),
            pltpu.VMEM_SHARED((NP, HID), f32),
        ] + [pltpu.SemaphoreType.DMA] * 8,
    )
    op_qa, op_aq = phase_b(
        ep_qa, ep_aq, ee_qa, ee_aq, den_qa, den_aq,
        hs_qa, hs_aq, z64)

    tc2 = pl.pallas_call(
        _tc2_body,
        grid=(GRID_N,),
        in_specs=[
            pl.BlockSpec((2, BN, HID), lambda i: (0, i, 0)),
            pl.BlockSpec((2, BN, HID), lambda i: (0, i, 0)),
            _full((1, HID)), _full((1, HID)),
            _full((2 * HID, NC_OUT)), _full((1, NC_OUT)),
        ],
        out_specs=pl.BlockSpec((BN, NC_OUT), lambda i: (i, 0)),
        out_shape=jax.ShapeDtypeStruct((NP, NC_OUT), f32),
    )
    preds = tc2(op_qa, op_aq, bias_qa_eff.reshape(1, HID),
                bias_aq_eff.reshape(1, HID), w_eff, b_out.reshape(1, NC_OUT))
    return (preds[:N], ew_qa, ew_aq)
